# baseline (speedup 1.0000x reference)
import functools

import jax
import jax.numpy as jnp
from jax import lax
from jax.experimental import pallas as pl
from jax.experimental.pallas import tpu as pltpu
from jax.experimental.pallas import tpu_sc as plsc

H = 512
N_OBS, N_AGT, N_GOAL = 4000, 5000, 1000
E_OA, E_AA, E_GA = 8000, 8000, 4000
E_TOT = E_OA + E_AA + E_GA
NC, NS, LN = 2, 16, 16
NW = NC * NS
E_OA_P, E_AA_P, E_GA_P = 8192, 8192, 4096
E_PAD = E_OA_P + E_AA_P + E_GA_P
E_PER_W = E_PAD // NW
GCB = 64
N_PAD = 5120
SCB = 512
NEG = -3.0e38


def _mm_body(a_ref, b_ref, o_ref):
    o_ref[...] = jnp.dot(a_ref[...], b_ref[...],
                         preferred_element_type=jnp.float32
                         ).astype(o_ref.dtype)


def _mm_bias_body(a_ref, b_ref, bias_ref, o_ref, *, relu):
    acc = jnp.dot(a_ref[...], b_ref[...], preferred_element_type=jnp.float32)
    acc = acc + bias_ref[...]
    if relu:
        acc = jnp.maximum(acc, 0.0)
    o_ref[...] = acc.astype(o_ref.dtype)


def _tc_mm(a, b, bias=None, relu=False, bm=512, out_dtype=jnp.float32):
    M, K = a.shape
    _, N = b.shape
    grid = (pl.cdiv(M, bm),)
    in_specs = [
        pl.BlockSpec((bm, K), lambda i: (i, 0)),
        pl.BlockSpec((K, N), lambda i: (0, 0)),
    ]
    args = [a, b]
    if bias is None:
        body = _mm_body
    else:
        body = functools.partial(_mm_bias_body, relu=relu)
        in_specs.append(pl.BlockSpec((1, N), lambda i: (0, 0)))
        args.append(bias.reshape(1, N))
    return pl.pallas_call(
        body,
        grid=grid,
        in_specs=in_specs,
        out_specs=pl.BlockSpec((bm, N), lambda i: (i, 0)),
        out_shape=jax.ShapeDtypeStruct((M, N), out_dtype),
    )(*args)


def _msg_body(g_ref, z_ref, epm_ref, epb_ref, w_ref, b_ref, o_ref):
    ep = jnp.dot(z_ref[...], epm_ref[...], preferred_element_type=jnp.float32)
    h = jnp.maximum(g_ref[...].astype(jnp.float32) + ep + epb_ref[...], 0.0)
    acc = jnp.dot(h.astype(jnp.bfloat16), w_ref[...],
                  preferred_element_type=jnp.float32) + b_ref[...]
    o_ref[...] = acc.T


def _tc_msg_t(g, z, epm, epb, w, b, bm=512):
    M = g.shape[0]
    return pl.pallas_call(
        _msg_body,
        grid=(pl.cdiv(M, bm),),
        in_specs=[
            pl.BlockSpec((bm, H), lambda i: (i, 0)),
            pl.BlockSpec((bm, H), lambda i: (i, 0)),
            pl.BlockSpec((H, H), lambda i: (0, 0)),
            pl.BlockSpec((1, H), lambda i: (0, 0)),
            pl.BlockSpec((H, H), lambda i: (0, 0)),
            pl.BlockSpec((1, H), lambda i: (0, 0)),
        ],
        out_specs=pl.BlockSpec((H, bm), lambda i: (0, i)),
        out_shape=jax.ShapeDtypeStruct((H, M), jnp.float32),
    )(g, z, epm, epb.reshape(1, H), w, b.reshape(1, H))


def _upd_body(xa_ref, agg_ref, floor_ref, o_ref):
    agg = agg_ref[...].T
    o_ref[...] = xa_ref[...] + jnp.maximum(agg, floor_ref[...])


def _tc_upd(xa, agg_t, floor, bn=512):
    M = xa.shape[0]
    return pl.pallas_call(
        _upd_body,
        grid=(pl.cdiv(M, bn),),
        in_specs=[
            pl.BlockSpec((bn, H), lambda i: (i, 0)),
            pl.BlockSpec((H, bn), lambda i: (0, i)),
            pl.BlockSpec((bn, 1), lambda i: (i, 0)),
        ],
        out_specs=pl.BlockSpec((bn, H), lambda i: (i, 0)),
        out_shape=jax.ShapeDtypeStruct((M, H), jnp.float32),
    )(xa, agg_t, floor.reshape(M, 1))


def _field_body(x_ref, act_ref, w1_ref, wa_ref, b1_ref, w2_ref, b2_ref, o_ref):
    h = jnp.dot(x_ref[...], w1_ref[...], preferred_element_type=jnp.float32)
    act = act_ref[...]
    h = h + act[:, 0:1] * wa_ref[0:1, :] + act[:, 1:2] * wa_ref[1:2, :]
    h = jnp.maximum(h + b1_ref[...], 0.0)
    o_ref[...] = jnp.sum(h * w2_ref[...], axis=1, keepdims=True) + b2_ref[...]


def _tc_field(x, act, w1, wa, b1, w2row, b2, bm=512):
    M = x.shape[0]
    return pl.pallas_call(
        _field_body,
        grid=(pl.cdiv(M, bm),),
        in_specs=[
            pl.BlockSpec((bm, H), lambda i: (i, 0)),
            pl.BlockSpec((bm, 2), lambda i: (i, 0)),
            pl.BlockSpec((H, H), lambda i: (0, 0)),
            pl.BlockSpec((2, H), lambda i: (0, 0)),
            pl.BlockSpec((1, H), lambda i: (0, 0)),
            pl.BlockSpec((1, H), lambda i: (0, 0)),
            pl.BlockSpec((1, 1), lambda i: (0, 0)),
        ],
        out_specs=pl.BlockSpec((bm, 1), lambda i: (i, 0)),
        out_shape=jax.ShapeDtypeStruct((M, 1), jnp.float32),
    )(x, act, w1, wa, b1.reshape(1, H), w2row.reshape(1, H),
      b2.reshape(1, 1))


def _sc_gather_body(dstp, srcp, dsti, srci, g_out,
                    idx_d, idx_s, dr0, dr1, sr0, sr1,
                    sg0, sg1, sg2, sg3, so0, so1):
    wid = lax.axis_index("s") * NC + lax.axis_index("c")
    ebase = wid * E_PER_W
    pltpu.sync_copy(dsti.at[pl.ds(ebase, E_PER_W)], idx_d)
    pltpu.sync_copy(srci.at[pl.ds(ebase, E_PER_W)], idx_s)
    dr, sr = (dr0, dr1), (sr0, sr1)
    sgd, sgs, so = (sg0, sg1), (sg2, sg3), (so0, so1)
    NCH = E_PER_W // GCB

    def issue(c, b):
        isl = pl.ds(c * GCB, GCB)
        h1 = pltpu.async_copy(dstp.at[idx_d.at[isl]], dr[b], sgd[b])
        h2 = pltpu.async_copy(srcp.at[idx_s.at[isl]], sr[b], sgs[b])
        return (h1, h2)

    def compute(b):
        drb, srb = dr[b], sr[b]

        def row(j, _):
            for k in range(H // (2 * LN)):
                sl = pl.ds(k * LN, LN)
                a = plsc.bitcast(drb[j, sl], jnp.bfloat16)
                s = plsc.bitcast(srb[j, sl], jnp.bfloat16)
                drb[j, sl] = plsc.bitcast(a + s, jnp.int32)
            return 0

        lax.fori_loop(0, GCB, row, 0)

    gh = {0: issue(0, 0)}
    wh = {}
    for c in range(NCH):
        b = c % 2
        if c + 1 < NCH:
            if c >= 1:
                wh.pop(1 - b).wait()
            gh[c + 1] = issue(c + 1, 1 - b)
        h1, h2 = gh.pop(c)
        h1.wait()
        h2.wait()
        compute(b)
        wh[b] = pltpu.async_copy(
            dr[b], g_out.at[pl.ds(ebase + c * GCB, GCB)], so[b])
    for b in list(wh):
        wh.pop(b).wait()


def _sc_gather(dstp, srcp, dsti, srci):
    mesh = plsc.VectorSubcoreMesh(core_axis_name="c", subcore_axis_name="s")
    fn = functools.partial(
        pl.kernel,
        mesh=mesh,
        compiler_params=pltpu.CompilerParams(needs_layout_passes=False),
        out_type=jax.ShapeDtypeStruct((E_PAD, H // 2), jnp.int32),
        scratch_types=[
            pltpu.VMEM((E_PER_W,), jnp.int32),
            pltpu.VMEM((E_PER_W,), jnp.int32),
            pltpu.VMEM((GCB, H // 2), jnp.int32),
            pltpu.VMEM((GCB, H // 2), jnp.int32),
            pltpu.VMEM((GCB, H // 2), jnp.int32),
            pltpu.VMEM((GCB, H // 2), jnp.int32),
            pltpu.SemaphoreType.DMA,
            pltpu.SemaphoreType.DMA,
            pltpu.SemaphoreType.DMA,
            pltpu.SemaphoreType.DMA,
            pltpu.SemaphoreType.DMA,
            pltpu.SemaphoreType.DMA,
        ],
    )(_sc_gather_body)
    return fn(dstp, srcp, dsti, srci)


_SEGS = ((0, E_OA_P, 0), (E_OA_P, E_OA_P + E_AA_P, 1),
         (E_OA_P + E_AA_P, E_PAD, 2))
NODES_PER_W = N_PAD // NW


def _sc_segmax_body(mt_hbm, dsti, aggt_hbm, floor_hbm,
                    acc, has0, has1, has2, dstc0, dstc1, mcol0, mcol1,
                    floorb, sd0, sd1, sm0, sm1):
    wid = lax.axis_index("s") * NC + lax.axis_index("c")
    f0 = wid * LN
    ones = jnp.ones((LN,), jnp.int32)
    iota = lax.iota(jnp.int32, LN)
    negrow = jnp.full((LN,), NEG, jnp.float32)
    zrow = jnp.zeros((LN,), jnp.int32)

    def init(n, _):
        for f in range(LN):
            acc[f, pl.ds(n * LN, LN)] = negrow
        return 0

    lax.fori_loop(0, N_PAD // LN, init, 0)

    def inith(q, _):
        sl = pl.ds(q * LN, LN)
        has0[sl] = zrow
        has1[sl] = zrow
        has2[sl] = zrow
        return 0

    lax.fori_loop(0, N_PAD // LN, inith, 0)

    dstc = (dstc0, dstc1)
    mcol = (mcol0, mcol1)
    sdd = (sd0, sd1)
    smm = (sm0, sm1)

    for (e0, e1, t) in _SEGS:
        has = (has0, has1, has2)[t]
        nch = (e1 - e0) // SCB

        def issue(c, b):
            off = e0 + c * SCB
            pltpu.async_copy(dsti.at[pl.ds(off, SCB)], dstc[b], sdd[b])
            pltpu.async_copy(
                mt_hbm.at[pl.ds(f0, LN), pl.ds(off, SCB)], mcol[b], smm[b])

        def wait(c, b):
            off = e0 + c * SCB
            pltpu.make_async_copy(
                dsti.at[pl.ds(off, SCB)], dstc[b], sdd[b]).wait()
            pltpu.make_async_copy(
                mt_hbm.at[pl.ds(f0, LN), pl.ds(off, SCB)],
                mcol[b], smm[b]).wait()

        def proc(ci, b):
            dstcb, mcolb = dstc[b], mcol[b]

            def group(q, _):
                j0 = q * LN
                dv = dstcb[pl.ds(j0, LN)]
                plsc.store_scatter(has, [dv], ones)
                _, lastm = plsc.scan_count(dv)
                nodup = jnp.all(lastm)

                def fast(x):
                    avs = []
                    for f in range(LN):
                        fs = jnp.full((LN,), f, jnp.int32)
                        avs.append(plsc.load_gather(acc, [fs, dv]))
                    mvs = [mcolb[f, pl.ds(j0, LN)] for f in range(LN)]
                    for f in range(LN):
                        fs = jnp.full((LN,), f, jnp.int32)
                        plsc.store_scatter(acc, [fs, dv],
                                           jnp.maximum(avs[f], mvs[f]))
                    return x

                def slow(x):
                    for j in range(LN):
                        dsp = jnp.full((LN,), dv[j], jnp.int32)
                        jsp = jnp.full((LN,), j0 + j, jnp.int32)
                        cur = plsc.load_gather(acc, [iota, dsp])
                        mv = plsc.load_gather(mcolb, [iota, jsp])
                        plsc.store_scatter(acc, [iota, dsp],
                                           jnp.maximum(cur, mv))
                    return x

                lax.cond(nodup, fast, slow, 0)
                return 0

            lax.fori_loop(0, SCB // LN, group, 0)

        issue(0, 0)

        def pipe(i, _):
            issue(2 * i + 1, 1)
            wait(2 * i, 0)
            proc(2 * i, 0)
            issue(2 * i + 2, 0)
            wait(2 * i + 1, 1)
            proc(2 * i + 1, 1)
            return 0

        lax.fori_loop(0, nch // 2 - 1, pipe, 0)
        issue(nch - 1, 1)
        wait(nch - 2, 0)
        proc(nch - 2, 0)
        wait(nch - 1, 1)
        proc(nch - 1, 1)

    pltpu.sync_copy(acc, aggt_hbm.at[pl.ds(f0, LN), :])

    n0 = wid * NODES_PER_W

    def fgrp(q, _):
        sl = pl.ds(n0 + q * LN, LN)
        hallv = (has0[sl] & has1[sl]) & has2[sl]
        floorb[pl.ds(q * LN, LN)] = jnp.where(
            hallv > 0, jnp.float32(NEG), jnp.float32(0.0))
        return 0

    lax.fori_loop(0, NODES_PER_W // LN, fgrp, 0)
    pltpu.sync_copy(floorb, floor_hbm.at[pl.ds(n0, NODES_PER_W)])


def _sc_segmax(m_t, dsti):
    mesh = plsc.VectorSubcoreMesh(core_axis_name="c", subcore_axis_name="s")
    fn = functools.partial(
        pl.kernel,
        mesh=mesh,
        compiler_params=pltpu.CompilerParams(needs_layout_passes=False),
        out_type=(jax.ShapeDtypeStruct((H, N_PAD), jnp.float32),
                  jax.ShapeDtypeStruct((N_PAD,), jnp.float32)),
        scratch_types=[
            pltpu.VMEM((LN, N_PAD), jnp.float32),
            pltpu.VMEM((N_PAD,), jnp.int32),
            pltpu.VMEM((N_PAD,), jnp.int32),
            pltpu.VMEM((N_PAD,), jnp.int32),
            pltpu.VMEM((SCB,), jnp.int32),
            pltpu.VMEM((SCB,), jnp.int32),
            pltpu.VMEM((LN, SCB), jnp.float32),
            pltpu.VMEM((LN, SCB), jnp.float32),
            pltpu.VMEM((NODES_PER_W,), jnp.float32),
            pltpu.SemaphoreType.DMA,
            pltpu.SemaphoreType.DMA,
            pltpu.SemaphoreType.DMA,
            pltpu.SemaphoreType.DMA,
        ],
    )(_sc_segmax_body)
    return fn(m_t, dsti)


def _pack_i32(x):
    M, N = x.shape
    return jax.lax.bitcast_convert_type(
        x.reshape(M, N // 2, 2), jnp.int32)


def _unpack_bf16(x):
    M, K = x.shape
    return jax.lax.bitcast_convert_type(x, jnp.bfloat16).reshape(M, 2 * K)


def kernel(x_obstacle, x_agent, x_goal, edge_index_oa, edge_index_aa,
           edge_index_ga, edge_attr_oa, edge_attr_aa, edge_attr_ga, action,
           W_embed, ee_W1, ee_b1, ee_W2, ee_b2,
           fx_W1_0, fx_b1_0, fx_W2_0, fx_b2_0,
           fx_W1_1, fx_b1_1, fx_W2_1, fx_b2_1,
           fld_W1, fld_b1, fld_W2, fld_b2):
    p_oa, p_aa, p_ga = E_OA_P - E_OA, E_AA_P - E_AA, E_GA_P - E_GA
    zi = jnp.zeros((), jnp.int32)
    src_pad = jnp.concatenate([
        edge_index_oa[0].astype(jnp.int32), jnp.zeros((p_oa,), jnp.int32),
        edge_index_aa[0].astype(jnp.int32) + N_OBS,
        jnp.zeros((p_aa,), jnp.int32),
        edge_index_ga[0].astype(jnp.int32) + (N_OBS + N_AGT),
        jnp.zeros((p_ga,), jnp.int32),
    ])
    dst_pad = jnp.concatenate([
        edge_index_oa[1].astype(jnp.int32),
        jnp.full((p_oa,), N_AGT, jnp.int32),
        edge_index_aa[1].astype(jnp.int32),
        jnp.full((p_aa,), N_AGT + 1, jnp.int32),
        edge_index_ga[1].astype(jnp.int32),
        jnp.full((p_ga,), N_AGT + 2, jnp.int32),
    ])

    z16 = jnp.zeros((1, 16), jnp.float32)
    ecat = jnp.concatenate([
        edge_attr_oa, jnp.tile(z16, (p_oa, 1)),
        edge_attr_aa, jnp.tile(z16, (p_aa, 1)),
        edge_attr_ga, jnp.tile(z16, (p_ga, 1)),
    ])
    xcat0 = jnp.concatenate([x_obstacle, x_agent, x_goal])

    xemb = _tc_mm(xcat0, W_embed)
    z = _tc_mm(ecat, ee_W1, ee_b1, relu=True,
               out_dtype=jnp.bfloat16)
    wstack = jnp.pad(jnp.concatenate([ee_W2, ee_b2[None, :]]),
                     ((0, 7), (0, 0)))

    xa = jnp.pad(xemb[N_OBS:N_OBS + N_AGT], ((0, N_PAD - N_AGT), (0, 0)))

    layer_params = ((fx_W1_0, fx_b1_0, fx_W2_0, fx_b2_0),
                    (fx_W1_1, fx_b1_1, fx_W2_1, fx_b2_1))
    for li, (W1, b1, W2, b2) in enumerate(layer_params):
        Wd, Ws, We = W1[:H], W1[H:2 * H], W1[2 * H:]
        epw = _tc_mm(wstack, We)
        ep_bias = epw[H] + b1

        if li == 0:
            xcat = xemb
        else:
            xcat = jnp.concatenate([
                xemb[:N_OBS], xa[:N_AGT], xemb[N_OBS + N_AGT:]])
        srcp = _tc_mm(xcat, Ws, out_dtype=jnp.bfloat16)
        dstp = _tc_mm(xa, Wd, out_dtype=jnp.bfloat16)

        g = _sc_gather(_pack_i32(dstp), _pack_i32(srcp),
                       dst_pad, src_pad)
        m_t = _tc_msg_t(_unpack_bf16(g), z,
                        epw[:H].astype(jnp.bfloat16), ep_bias,
                        W2.astype(jnp.bfloat16), b2)
        agg_t, floor = _sc_segmax(m_t, dst_pad)
        xa = _tc_upd(xa, agg_t, floor)

    field = _tc_field(xa[:N_AGT], action, fld_W1[:H], fld_W1[H:],
                      fld_b1, fld_W2[:, 0], fld_b2)
    return field[:, 0]


# speedup vs baseline: 2.2543x; 2.2543x over previous
import functools

import jax
import jax.numpy as jnp
from jax import lax
from jax.experimental import pallas as pl
from jax.experimental.pallas import tpu as pltpu
from jax.experimental.pallas import tpu_sc as plsc

H = 512
N_OBS, N_AGT, N_GOAL = 4000, 5000, 1000
E_OA, E_AA, E_GA = 8000, 8000, 4000
E_TOT = E_OA + E_AA + E_GA
NC, NS, LN = 2, 16, 16
NW = NC * NS
E_OA_P, E_AA_P, E_GA_P = 8192, 8192, 4096
E_PAD = E_OA_P + E_AA_P + E_GA_P
E_PER_W = E_PAD // NW
GCB = 64
N_PAD = 5120
SCB = 512
NEG = -3.0e38


def _mm_body(a_ref, b_ref, o_ref):
    o_ref[...] = jnp.dot(a_ref[...], b_ref[...],
                         preferred_element_type=jnp.float32
                         ).astype(o_ref.dtype)


def _mm_pack_body(a_ref, b_ref, o_ref):
    acc = jnp.dot(a_ref[...], b_ref[...], preferred_element_type=jnp.float32)
    o_ref[...] = _pack_halves(acc)


def _tc_mm_packed(a, b, bm=512):
    M, K = a.shape
    _, N = b.shape
    return pl.pallas_call(
        _mm_pack_body,
        grid=(pl.cdiv(M, bm),),
        in_specs=[
            pl.BlockSpec((bm, K), lambda i: (i, 0)),
            pl.BlockSpec((K, N), lambda i: (0, 0)),
        ],
        out_specs=pl.BlockSpec((bm, N // 2), lambda i: (i, 0)),
        out_shape=jax.ShapeDtypeStruct((M, N // 2), jnp.int32),
    )(a, b)


def _mm_bias_body(a_ref, b_ref, bias_ref, o_ref, *, relu):
    acc = jnp.dot(a_ref[...], b_ref[...], preferred_element_type=jnp.float32)
    acc = acc + bias_ref[...]
    if relu:
        acc = jnp.maximum(acc, 0.0)
    o_ref[...] = acc.astype(o_ref.dtype)


def _tc_mm(a, b, bias=None, relu=False, bm=512, out_dtype=jnp.float32):
    M, K = a.shape
    _, N = b.shape
    grid = (pl.cdiv(M, bm),)
    in_specs = [
        pl.BlockSpec((bm, K), lambda i: (i, 0)),
        pl.BlockSpec((K, N), lambda i: (0, 0)),
    ]
    args = [a, b]
    if bias is None:
        body = _mm_body
    else:
        body = functools.partial(_mm_bias_body, relu=relu)
        in_specs.append(pl.BlockSpec((1, N), lambda i: (0, 0)))
        args.append(bias.reshape(1, N))
    return pl.pallas_call(
        body,
        grid=grid,
        in_specs=in_specs,
        out_specs=pl.BlockSpec((bm, N), lambda i: (i, 0)),
        out_shape=jax.ShapeDtypeStruct((M, N), out_dtype),
    )(*args)


def _msg_body(g_ref, z_ref, epm_ref, epb_ref, w_ref, b_ref, o_ref):
    ep = jnp.dot(z_ref[...], epm_ref[...], preferred_element_type=jnp.float32)
    g = _unpack_halves(g_ref[...])
    h = jnp.maximum(g + ep + epb_ref[...], 0.0)
    acc = jnp.dot(h.astype(jnp.bfloat16), w_ref[...],
                  preferred_element_type=jnp.float32) + b_ref[...]
    o_ref[...] = acc.T


def _tc_msg_t(g, z, epm, epb, w, b, bm=512):
    M = g.shape[0]
    return pl.pallas_call(
        _msg_body,
        grid=(pl.cdiv(M, bm),),
        in_specs=[
            pl.BlockSpec((bm, H // 2), lambda i: (i, 0)),
            pl.BlockSpec((bm, H), lambda i: (i, 0)),
            pl.BlockSpec((H, H), lambda i: (0, 0)),
            pl.BlockSpec((1, H), lambda i: (0, 0)),
            pl.BlockSpec((H, H), lambda i: (0, 0)),
            pl.BlockSpec((1, H), lambda i: (0, 0)),
        ],
        out_specs=pl.BlockSpec((H, bm), lambda i: (0, i)),
        out_shape=jax.ShapeDtypeStruct((H, M), jnp.float32),
    )(g, z, epm, epb.reshape(1, H), w, b.reshape(1, H))


def _upd_body(xa_ref, agg_ref, floor_ref, o_ref):
    agg = agg_ref[...].T
    o_ref[...] = xa_ref[...] + jnp.maximum(agg, floor_ref[...])


def _tc_upd(xa, agg_t, floor, bn=512):
    M = xa.shape[0]
    return pl.pallas_call(
        _upd_body,
        grid=(pl.cdiv(M, bn),),
        in_specs=[
            pl.BlockSpec((bn, H), lambda i: (i, 0)),
            pl.BlockSpec((H, bn), lambda i: (0, i)),
            pl.BlockSpec((bn, 1), lambda i: (i, 0)),
        ],
        out_specs=pl.BlockSpec((bn, H), lambda i: (i, 0)),
        out_shape=jax.ShapeDtypeStruct((M, H), jnp.float32),
    )(xa, agg_t, floor.reshape(M, 1))


def _field_body(x_ref, act_ref, w1_ref, wa_ref, b1_ref, w2_ref, b2_ref, o_ref):
    h = jnp.dot(x_ref[...], w1_ref[...], preferred_element_type=jnp.float32)
    act = act_ref[...]
    h = h + act[:, 0:1] * wa_ref[0:1, :] + act[:, 1:2] * wa_ref[1:2, :]
    h = jnp.maximum(h + b1_ref[...], 0.0)
    o_ref[...] = jnp.sum(h * w2_ref[...], axis=1, keepdims=True) + b2_ref[...]


def _tc_field(x, act, w1, wa, b1, w2row, b2, bm=512):
    M = x.shape[0]
    return pl.pallas_call(
        _field_body,
        grid=(pl.cdiv(M, bm),),
        in_specs=[
            pl.BlockSpec((bm, H), lambda i: (i, 0)),
            pl.BlockSpec((bm, 2), lambda i: (i, 0)),
            pl.BlockSpec((H, H), lambda i: (0, 0)),
            pl.BlockSpec((2, H), lambda i: (0, 0)),
            pl.BlockSpec((1, H), lambda i: (0, 0)),
            pl.BlockSpec((1, H), lambda i: (0, 0)),
            pl.BlockSpec((1, 1), lambda i: (0, 0)),
        ],
        out_specs=pl.BlockSpec((bm, 1), lambda i: (i, 0)),
        out_shape=jax.ShapeDtypeStruct((M, 1), jnp.float32),
    )(x, act, w1, wa, b1.reshape(1, H), w2row.reshape(1, H),
      b2.reshape(1, 1))


def _sc_gather_body(dstp, srcp, dsti, srci, g_out,
                    idx_d, idx_s, dr0, dr1, sr0, sr1,
                    sg0, sg1, sg2, sg3, so0, so1):
    wid = lax.axis_index("s") * NC + lax.axis_index("c")
    ebase = wid * E_PER_W
    pltpu.sync_copy(dsti.at[pl.ds(ebase, E_PER_W)], idx_d)
    pltpu.sync_copy(srci.at[pl.ds(ebase, E_PER_W)], idx_s)
    dr, sr = (dr0, dr1), (sr0, sr1)
    sgd, sgs, so = (sg0, sg1), (sg2, sg3), (so0, so1)
    NCH = E_PER_W // GCB

    def issue(c, b):
        isl = pl.ds(c * GCB, GCB)
        h1 = pltpu.async_copy(dstp.at[idx_d.at[isl]], dr[b], sgd[b])
        h2 = pltpu.async_copy(srcp.at[idx_s.at[isl]], sr[b], sgs[b])
        return (h1, h2)

    def compute(b):
        drb, srb = dr[b], sr[b]

        def row(j, _):
            for k in range(H // (2 * LN)):
                sl = pl.ds(k * LN, LN)
                a = plsc.bitcast(drb[j, sl], jnp.bfloat16)
                s = plsc.bitcast(srb[j, sl], jnp.bfloat16)
                drb[j, sl] = plsc.bitcast(a + s, jnp.int32)
            return 0

        lax.fori_loop(0, GCB, row, 0)

    gh = {0: issue(0, 0)}
    wh = {}
    for c in range(NCH):
        b = c % 2
        if c + 1 < NCH:
            if c >= 1:
                wh.pop(1 - b).wait()
            gh[c + 1] = issue(c + 1, 1 - b)
        h1, h2 = gh.pop(c)
        h1.wait()
        h2.wait()
        compute(b)
        wh[b] = pltpu.async_copy(
            dr[b], g_out.at[pl.ds(ebase + c * GCB, GCB)], so[b])
    for b in list(wh):
        wh.pop(b).wait()


def _sc_gather(dstp, srcp, dsti, srci):
    mesh = plsc.VectorSubcoreMesh(core_axis_name="c", subcore_axis_name="s")
    fn = functools.partial(
        pl.kernel,
        mesh=mesh,
        compiler_params=pltpu.CompilerParams(needs_layout_passes=False),
        out_type=jax.ShapeDtypeStruct((E_PAD, H // 2), jnp.int32),
        scratch_types=[
            pltpu.VMEM((E_PER_W,), jnp.int32),
            pltpu.VMEM((E_PER_W,), jnp.int32),
            pltpu.VMEM((GCB, H // 2), jnp.int32),
            pltpu.VMEM((GCB, H // 2), jnp.int32),
            pltpu.VMEM((GCB, H // 2), jnp.int32),
            pltpu.VMEM((GCB, H // 2), jnp.int32),
            pltpu.SemaphoreType.DMA,
            pltpu.SemaphoreType.DMA,
            pltpu.SemaphoreType.DMA,
            pltpu.SemaphoreType.DMA,
            pltpu.SemaphoreType.DMA,
            pltpu.SemaphoreType.DMA,
        ],
    )(_sc_gather_body)
    return fn(dstp, srcp, dsti, srci)


_SEGS = ((0, E_OA_P, 0), (E_OA_P, E_OA_P + E_AA_P, 1),
         (E_OA_P + E_AA_P, E_PAD, 2))
NODES_PER_W = N_PAD // NW


def _sc_segmax_body(mt_hbm, dsti, aggt_hbm, floor_hbm,
                    acc, has0, has1, has2, dstc0, dstc1, mcol0, mcol1,
                    floorb, sd0, sd1, sm0, sm1):
    wid = lax.axis_index("s") * NC + lax.axis_index("c")
    f0 = wid * LN
    ones = jnp.ones((LN,), jnp.int32)
    iota = lax.iota(jnp.int32, LN)
    negrow = jnp.full((LN,), NEG, jnp.float32)
    zrow = jnp.zeros((LN,), jnp.int32)

    def init(n, _):
        for f in range(LN):
            acc[f, pl.ds(n * LN, LN)] = negrow
        return 0

    lax.fori_loop(0, N_PAD // LN, init, 0)

    def inith(q, _):
        sl = pl.ds(q * LN, LN)
        has0[sl] = zrow
        has1[sl] = zrow
        has2[sl] = zrow
        return 0

    lax.fori_loop(0, N_PAD // LN, inith, 0)

    dstc = (dstc0, dstc1)
    mcol = (mcol0, mcol1)
    sdd = (sd0, sd1)
    smm = (sm0, sm1)

    for (e0, e1, t) in _SEGS:
        has = (has0, has1, has2)[t]
        nch = (e1 - e0) // SCB

        def issue(c, b):
            off = e0 + c * SCB
            pltpu.async_copy(dsti.at[pl.ds(off, SCB)], dstc[b], sdd[b])
            pltpu.async_copy(
                mt_hbm.at[pl.ds(f0, LN), pl.ds(off, SCB)], mcol[b], smm[b])

        def wait(c, b):
            off = e0 + c * SCB
            pltpu.make_async_copy(
                dsti.at[pl.ds(off, SCB)], dstc[b], sdd[b]).wait()
            pltpu.make_async_copy(
                mt_hbm.at[pl.ds(f0, LN), pl.ds(off, SCB)],
                mcol[b], smm[b]).wait()

        def proc(ci, b):
            dstcb, mcolb = dstc[b], mcol[b]

            def group(q, _):
                j0 = q * LN
                dv = dstcb[pl.ds(j0, LN)]
                plsc.store_scatter(has, [dv], ones)
                _, lastm = plsc.scan_count(dv)
                nodup = jnp.all(lastm)

                def fast(x):
                    avs = []
                    for f in range(LN):
                        fs = jnp.full((LN,), f, jnp.int32)
                        avs.append(plsc.load_gather(acc, [fs, dv]))
                    mvs = [mcolb[f, pl.ds(j0, LN)] for f in range(LN)]
                    for f in range(LN):
                        fs = jnp.full((LN,), f, jnp.int32)
                        plsc.store_scatter(acc, [fs, dv],
                                           jnp.maximum(avs[f], mvs[f]))
                    return x

                def slow(x):
                    for j in range(LN):
                        dsp = jnp.full((LN,), dv[j], jnp.int32)
                        jsp = jnp.full((LN,), j0 + j, jnp.int32)
                        cur = plsc.load_gather(acc, [iota, dsp])
                        mv = plsc.load_gather(mcolb, [iota, jsp])
                        plsc.store_scatter(acc, [iota, dsp],
                                           jnp.maximum(cur, mv))
                    return x

                lax.cond(nodup, fast, slow, 0)
                return 0

            lax.fori_loop(0, SCB // LN, group, 0)

        issue(0, 0)

        def pipe(i, _):
            issue(2 * i + 1, 1)
            wait(2 * i, 0)
            proc(2 * i, 0)
            issue(2 * i + 2, 0)
            wait(2 * i + 1, 1)
            proc(2 * i + 1, 1)
            return 0

        lax.fori_loop(0, nch // 2 - 1, pipe, 0)
        issue(nch - 1, 1)
        wait(nch - 2, 0)
        proc(nch - 2, 0)
        wait(nch - 1, 1)
        proc(nch - 1, 1)

    pltpu.sync_copy(acc, aggt_hbm.at[pl.ds(f0, LN), :])

    n0 = wid * NODES_PER_W

    def fgrp(q, _):
        sl = pl.ds(n0 + q * LN, LN)
        hallv = (has0[sl] & has1[sl]) & has2[sl]
        floorb[pl.ds(q * LN, LN)] = jnp.where(
            hallv > 0, jnp.float32(NEG), jnp.float32(0.0))
        return 0

    lax.fori_loop(0, NODES_PER_W // LN, fgrp, 0)
    pltpu.sync_copy(floorb, floor_hbm.at[pl.ds(n0, NODES_PER_W)])


def _sc_segmax(m_t, dsti):
    mesh = plsc.VectorSubcoreMesh(core_axis_name="c", subcore_axis_name="s")
    fn = functools.partial(
        pl.kernel,
        mesh=mesh,
        compiler_params=pltpu.CompilerParams(needs_layout_passes=False),
        out_type=(jax.ShapeDtypeStruct((H, N_PAD), jnp.float32),
                  jax.ShapeDtypeStruct((N_PAD,), jnp.float32)),
        scratch_types=[
            pltpu.VMEM((LN, N_PAD), jnp.float32),
            pltpu.VMEM((N_PAD,), jnp.int32),
            pltpu.VMEM((N_PAD,), jnp.int32),
            pltpu.VMEM((N_PAD,), jnp.int32),
            pltpu.VMEM((SCB,), jnp.int32),
            pltpu.VMEM((SCB,), jnp.int32),
            pltpu.VMEM((LN, SCB), jnp.float32),
            pltpu.VMEM((LN, SCB), jnp.float32),
            pltpu.VMEM((NODES_PER_W,), jnp.float32),
            pltpu.SemaphoreType.DMA,
            pltpu.SemaphoreType.DMA,
            pltpu.SemaphoreType.DMA,
            pltpu.SemaphoreType.DMA,
        ],
    )(_sc_segmax_body)
    return fn(m_t, dsti)


def _pack_halves(acc):
    n2 = acc.shape[1] // 2
    lo = acc[:, :n2].astype(jnp.bfloat16).astype(jnp.float32)
    hi = acc[:, n2:].astype(jnp.bfloat16).astype(jnp.float32)
    lo_w = lax.shift_right_logical(
        lax.bitcast_convert_type(lo, jnp.int32), 16)
    hi_w = lax.bitcast_convert_type(hi, jnp.int32) & jnp.int32(-65536)
    return lo_w | hi_w


def _unpack_halves(w):
    lo = lax.bitcast_convert_type(lax.shift_left(w, 16), jnp.float32)
    hi = lax.bitcast_convert_type(w & jnp.int32(-65536), jnp.float32)
    return jnp.concatenate([lo, hi], axis=1)


def kernel(x_obstacle, x_agent, x_goal, edge_index_oa, edge_index_aa,
           edge_index_ga, edge_attr_oa, edge_attr_aa, edge_attr_ga, action,
           W_embed, ee_W1, ee_b1, ee_W2, ee_b2,
           fx_W1_0, fx_b1_0, fx_W2_0, fx_b2_0,
           fx_W1_1, fx_b1_1, fx_W2_1, fx_b2_1,
           fld_W1, fld_b1, fld_W2, fld_b2):
    p_oa, p_aa, p_ga = E_OA_P - E_OA, E_AA_P - E_AA, E_GA_P - E_GA
    zi = jnp.zeros((), jnp.int32)
    src_pad = jnp.concatenate([
        edge_index_oa[0].astype(jnp.int32), jnp.zeros((p_oa,), jnp.int32),
        edge_index_aa[0].astype(jnp.int32) + N_OBS,
        jnp.zeros((p_aa,), jnp.int32),
        edge_index_ga[0].astype(jnp.int32) + (N_OBS + N_AGT),
        jnp.zeros((p_ga,), jnp.int32),
    ])
    dst_pad = jnp.concatenate([
        edge_index_oa[1].astype(jnp.int32),
        jnp.full((p_oa,), N_AGT, jnp.int32),
        edge_index_aa[1].astype(jnp.int32),
        jnp.full((p_aa,), N_AGT + 1, jnp.int32),
        edge_index_ga[1].astype(jnp.int32),
        jnp.full((p_ga,), N_AGT + 2, jnp.int32),
    ])

    z16 = jnp.zeros((1, 16), jnp.float32)
    ecat = jnp.concatenate([
        edge_attr_oa, jnp.tile(z16, (p_oa, 1)),
        edge_attr_aa, jnp.tile(z16, (p_aa, 1)),
        edge_attr_ga, jnp.tile(z16, (p_ga, 1)),
    ])
    xcat0 = jnp.concatenate([x_obstacle, x_agent, x_goal])

    xemb = _tc_mm(xcat0, W_embed)
    z = _tc_mm(ecat, ee_W1, ee_b1, relu=True,
               out_dtype=jnp.bfloat16)
    wstack = jnp.pad(jnp.concatenate([ee_W2, ee_b2[None, :]]),
                     ((0, 7), (0, 0)))

    xa = jnp.pad(xemb[N_OBS:N_OBS + N_AGT], ((0, N_PAD - N_AGT), (0, 0)))

    layer_params = ((fx_W1_0, fx_b1_0, fx_W2_0, fx_b2_0),
                    (fx_W1_1, fx_b1_1, fx_W2_1, fx_b2_1))
    for li, (W1, b1, W2, b2) in enumerate(layer_params):
        Wd, Ws, We = W1[:H], W1[H:2 * H], W1[2 * H:]
        epw = _tc_mm(wstack, We)
        ep_bias = epw[H] + b1

        if li == 0:
            xcat = xemb
        else:
            xcat = jnp.concatenate([
                xemb[:N_OBS], xa[:N_AGT], xemb[N_OBS + N_AGT:]])
        srcp = _tc_mm_packed(xcat, Ws)
        dstp = _tc_mm_packed(xa, Wd)

        g = _sc_gather(dstp, srcp, dst_pad, src_pad)
        m_t = _tc_msg_t(g, z, epw[:H].astype(jnp.bfloat16), ep_bias,
                        W2.astype(jnp.bfloat16), b2)
        agg_t, floor = _sc_segmax(m_t, dst_pad)
        xa = _tc_upd(xa, agg_t, floor)

    field = _tc_field(xa[:N_AGT], action, fld_W1[:H], fld_W1[H:],
                      fld_b1, fld_W2[:, 0], fld_b2)
    return field[:, 0]
